# Initial kernel scaffold; baseline (speedup 1.0000x reference)
import jax
import jax.numpy as jnp
from jax.experimental import pallas as pl


def kernel(x, Ws1, Ws2, W1, W2, Wr, rb):
    raise NotImplementedError("write your pallas kernel here")


# speedup vs baseline: 1.2176x; 1.2176x over previous
import functools

import jax
import jax.numpy as jnp
from jax.experimental import pallas as pl
from jax.experimental.pallas import tpu as pltpu

T = 2048
D = 1024
F = 1024
E = 8
K = 2
TM = 256


def _moe_kernel(x_ref, Ws1_ref, Ws2_ref, W1_ref, W2_ref, Wr_ref, rb_ref,
                out_ref, cnt_ref):
    t = pl.program_id(0)
    e = pl.program_id(1)

    xt = x_ref[...]

    logits = jnp.dot(xt, Wr_ref[...], preferred_element_type=jnp.float32)
    logits = logits - jnp.max(logits, axis=-1, keepdims=True)
    ex = jnp.exp(logits)
    scores = ex / jnp.sum(ex, axis=-1, keepdims=True)
    sel = scores + rb_ref[...]
    iota_e = jax.lax.broadcasted_iota(jnp.int32, (TM, E), 1)

    m1 = jnp.max(sel, axis=-1, keepdims=True)
    i1 = jnp.min(jnp.where(sel == m1, iota_e, E), axis=-1, keepdims=True)
    sel2 = jnp.where(iota_e == i1, -jnp.inf, sel)
    m2 = jnp.max(sel2, axis=-1, keepdims=True)
    i2 = jnp.min(jnp.where(sel2 == m2, iota_e, E), axis=-1, keepdims=True)

    g1 = jnp.sum(jnp.where(iota_e == i1, scores, 0.0), axis=-1, keepdims=True)
    g2 = jnp.sum(jnp.where(iota_e == i2, scores, 0.0), axis=-1, keepdims=True)
    denom = g1 + g2 + 1e-9
    w1 = g1 / denom
    w2 = g2 / denom

    w_e = jnp.where(i1 == e, w1, 0.0) + jnp.where(i2 == e, w2, 0.0)

    h = jnp.dot(xt, W1_ref[0], preferred_element_type=jnp.float32)
    h = h * jax.nn.sigmoid(h)
    contrib = jnp.dot(h, W2_ref[0], preferred_element_type=jnp.float32) * w_e

    @pl.when(e == 0)
    def _init():
        hs = jnp.dot(xt, Ws1_ref[...], preferred_element_type=jnp.float32)
        hs = hs * jax.nn.sigmoid(hs)
        shared = jnp.dot(hs, Ws2_ref[...], preferred_element_type=jnp.float32)
        out_ref[...] = shared + contrib
        tile_cnt = (jnp.sum((iota_e == i1).astype(jnp.int32), axis=0, keepdims=True)
                    + jnp.sum((iota_e == i2).astype(jnp.int32), axis=0, keepdims=True))

        @pl.when(t == 0)
        def _():
            cnt_ref[...] = tile_cnt

        @pl.when(t != 0)
        def _():
            cnt_ref[...] += tile_cnt

    @pl.when(e != 0)
    def _acc():
        out_ref[...] += contrib


@functools.partial(jax.jit, static_argnames=())
def kernel(x, Ws1, Ws2, W1, W2, Wr, rb):
    b, s, d = x.shape
    xf = x.reshape(s, d)
    rb2 = rb.reshape(1, E)

    grid = (T // TM, E)
    out, cnt = pl.pallas_call(
        _moe_kernel,
        grid=grid,
        in_specs=[
            pl.BlockSpec((TM, D), lambda t, e: (t, 0)),
            pl.BlockSpec((D, F), lambda t, e: (0, 0)),
            pl.BlockSpec((F, D), lambda t, e: (0, 0)),
            pl.BlockSpec((1, D, F), lambda t, e: (e, 0, 0)),
            pl.BlockSpec((1, F, D), lambda t, e: (e, 0, 0)),
            pl.BlockSpec((D, E), lambda t, e: (0, 0)),
            pl.BlockSpec((1, E), lambda t, e: (0, 0)),
        ],
        out_specs=[
            pl.BlockSpec((TM, D), lambda t, e: (t, 0)),
            pl.BlockSpec((1, E), lambda t, e: (0, 0)),
        ],
        out_shape=[
            jax.ShapeDtypeStruct((T, D), jnp.float32),
            jax.ShapeDtypeStruct((1, E), jnp.int32),
        ],
        compiler_params=pltpu.CompilerParams(
            dimension_semantics=("arbitrary", "arbitrary"),
        ),
    )(xf, Ws1, Ws2, W1, W2, Wr, rb2)

    return out.reshape(b, s, d), cnt.reshape(E)
